# Initial kernel scaffold; baseline (speedup 1.0000x reference)
import functools

import jax
import jax.numpy as jnp
from jax import lax
from jax.experimental import pallas as pl
from jax.experimental.pallas import tpu as pltpu
from jax.experimental.pallas import tpu_sc as plsc

F32 = jnp.float32

NC = 2
NS = 16
NW = NC * NS
CHUNK = 128
DEG_W = 16


def _pad_rows(n):
    per_tile = -(-n // (NS * CHUNK)) * CHUNK
    return NS * per_tile, per_tile


def _sc_deg(n_nodes, k_chunks):
    rows, per_tile = _pad_rows(n_nodes + 1)
    nblk = per_tile // CHUNK
    mesh = plsc.VectorSubcoreMesh(core_axis_name="c", subcore_axis_name="s")

    @functools.partial(
        pl.kernel, mesh=mesh,
        out_type=jax.ShapeDtypeStruct((NC, rows, DEG_W), F32),
        scratch_types=[
            pltpu.VMEM((k_chunks, CHUNK), jnp.int32),
            pltpu.VMEM((CHUNK, DEG_W), F32),
            pltpu.VMEM_SHARED((rows, DEG_W), F32),
        ],
    )
    def deg_kernel(dst_hbm, out_hbm, dst_v, ones_v, acc):
        c = lax.axis_index("c")
        s = lax.axis_index("s")

        def zero_row(i, _):
            ones_v[i, :] = jnp.zeros((DEG_W,), F32)
            return 0
        lax.fori_loop(0, CHUNK, zero_row, 0)
        for b in range(nblk):
            pltpu.sync_copy(ones_v, acc.at[pl.ds(s * per_tile + b * CHUNK, CHUNK)])

        def fill(i, _):
            ones_v[i, :] = jnp.ones((DEG_W,), F32)
            return 0
        lax.fori_loop(0, CHUNK, fill, 0)

        pltpu.sync_copy(dst_hbm.at[c, s], dst_v)
        plsc.subcore_barrier()

        def body(j, _):
            pltpu.sync_copy(ones_v, acc.at[dst_v.at[j]], add=True)
            return 0
        lax.fori_loop(0, k_chunks, body, 0)

        plsc.subcore_barrier()
        pltpu.sync_copy(acc.at[pl.ds(s * per_tile, per_tile)],
                        out_hbm.at[c, pl.ds(s * per_tile, per_tile)])

    return deg_kernel


def _sc_agg(n_nodes, d_cols, k_chunks):
    rows, per_tile = _pad_rows(n_nodes + 1)
    nblk = per_tile // CHUNK
    mesh = plsc.VectorSubcoreMesh(core_axis_name="c", subcore_axis_name="s")

    @functools.partial(
        pl.kernel, mesh=mesh,
        out_type=jax.ShapeDtypeStruct((NC, rows, d_cols), F32),
        scratch_types=[
            pltpu.VMEM((k_chunks, CHUNK), jnp.int32),
            pltpu.VMEM((k_chunks, CHUNK), jnp.int32),
            pltpu.VMEM((CHUNK, d_cols), F32),
            pltpu.VMEM_SHARED((rows, d_cols), F32),
            pltpu.SemaphoreType.DMA,
        ],
    )
    def agg_kernel(y_hbm, src_hbm, dst_hbm, out_hbm,
                   src_v, dst_v, rows_v, acc, sem):
        c = lax.axis_index("c")
        s = lax.axis_index("s")

        def zero_row(i, _):
            for cc in range(d_cols // 16):
                rows_v[i, pl.ds(cc * 16, 16)] = jnp.zeros((16,), F32)
            return 0
        lax.fori_loop(0, CHUNK, zero_row, 0)
        for b in range(nblk):
            pltpu.sync_copy(rows_v, acc.at[pl.ds(s * per_tile + b * CHUNK, CHUNK)])

        pltpu.sync_copy(src_hbm.at[c, s], src_v)
        pltpu.sync_copy(dst_hbm.at[c, s], dst_v)
        plsc.subcore_barrier()

        def body(j, _):
            pltpu.async_copy(y_hbm.at[src_v.at[j]], rows_v, sem).wait()
            pltpu.sync_copy(rows_v, acc.at[dst_v.at[j]], add=True)
            return 0
        lax.fori_loop(0, k_chunks, body, 0)

        plsc.subcore_barrier()
        pltpu.sync_copy(acc.at[pl.ds(s * per_tile, per_tile)],
                        out_hbm.at[c, pl.ds(s * per_tile, per_tile)])

    return agg_kernel


_SELU_ALPHA = 1.6732632423543772
_SELU_SCALE = 1.0507009873554805


def _dinv_from_deg(degp_ref):
    deg = degp_ref[0, :, 0:1] + degp_ref[1, :, 0:1] + 1.0
    return lax.rsqrt(deg)


def _tc_y1_body(degp_ref, x_ref, w1_ref, y1_ref):
    dinv = _dinv_from_deg(degp_ref)
    xw = jnp.dot(x_ref[...], w1_ref[...], preferred_element_type=F32)
    y1_ref[...] = dinv * xw


def _tc_mid_body(degp_ref, aggp_ref, y1_ref, b1_ref, w2_ref, y2_ref):
    dinv = _dinv_from_deg(degp_ref)
    su = aggp_ref[0] + aggp_ref[1] + y1_ref[...]
    pre = dinv * su + b1_ref[...][None, :]
    h = _SELU_SCALE * jnp.where(pre > 0, pre, _SELU_ALPHA * jnp.expm1(pre))
    y2_ref[...] = dinv * jnp.dot(h, w2_ref[...], preferred_element_type=F32)


def _tc_out_body(degp_ref, aggp_ref, y2_ref, b2_ref, o_ref):
    dinv = _dinv_from_deg(degp_ref)
    z = dinv * (aggp_ref[0] + aggp_ref[1] + y2_ref[...]) + b2_ref[...][None, :]
    m = jnp.max(z, axis=1, keepdims=True)
    e = jnp.exp(z - m)
    o_ref[...] = e / jnp.sum(e, axis=1, keepdims=True)


def kernel(x, edge_index, W1, b1, W2, b2):
    n, d_in = x.shape
    h_dim = W1.shape[1]
    o_dim = W2.shape[1]
    e = edge_index.shape[1]

    k_chunks = -(-e // (NW * CHUNK))
    e_pad = NW * k_chunks * CHUNK
    src = jnp.concatenate(
        [edge_index[0], jnp.zeros((e_pad - e,), jnp.int32)]).reshape(
            NC, NS, k_chunks, CHUNK)
    dst = jnp.concatenate(
        [edge_index[1], jnp.full((e_pad - e,), n, jnp.int32)]).reshape(
            NC, NS, k_chunks, CHUNK)

    degp = _sc_deg(n, k_chunks)(dst)

    rblk = 2000
    grid = (n // rblk,)
    degp_spec = pl.BlockSpec((NC, rblk, DEG_W), lambda i: (0, i, 0))
    aggp_spec = lambda d: pl.BlockSpec((NC, rblk, d), lambda i: (0, i, 0))
    full = lambda *shape: pl.BlockSpec(shape, lambda i: (0,) * len(shape))

    y1 = pl.pallas_call(
        _tc_y1_body,
        grid=grid,
        in_specs=[degp_spec,
                  pl.BlockSpec((rblk, d_in), lambda i: (i, 0)),
                  full(d_in, h_dim)],
        out_specs=pl.BlockSpec((rblk, h_dim), lambda i: (i, 0)),
        out_shape=jax.ShapeDtypeStruct((n, h_dim), F32),
    )(degp, x, W1)

    agg1p = _sc_agg(n, h_dim, k_chunks)(y1, src, dst)

    y2 = pl.pallas_call(
        _tc_mid_body,
        grid=grid,
        in_specs=[degp_spec, aggp_spec(h_dim),
                  pl.BlockSpec((rblk, h_dim), lambda i: (i, 0)),
                  full(h_dim), full(h_dim, o_dim)],
        out_specs=pl.BlockSpec((rblk, o_dim), lambda i: (i, 0)),
        out_shape=jax.ShapeDtypeStruct((n, o_dim), F32),
    )(degp, agg1p, y1, b1, W2)

    agg2p = _sc_agg(n, o_dim, k_chunks)(y2, src, dst)

    out = pl.pallas_call(
        _tc_out_body,
        grid=grid,
        in_specs=[degp_spec, aggp_spec(o_dim),
                  pl.BlockSpec((rblk, o_dim), lambda i: (i, 0)),
                  full(o_dim)],
        out_specs=pl.BlockSpec((rblk, o_dim), lambda i: (i, 0)),
        out_shape=jax.ShapeDtypeStruct((n, o_dim), F32),
    )(degp, agg2p, y2, b2)

    return out


# speedup vs baseline: 13.8209x; 13.8209x over previous
import functools

import jax
import jax.numpy as jnp
from jax import lax
from jax.experimental import pallas as pl
from jax.experimental.pallas import tpu as pltpu
from jax.experimental.pallas import tpu_sc as plsc

F32 = jnp.float32

NC = 2
NS = 16
NW = NC * NS
CHUNK = 128
DEG_W = 16


def _pad_rows(n):
    per_tile = -(-n // (NS * CHUNK)) * CHUNK
    return NS * per_tile, per_tile


def _sc_deg(n_nodes, k_chunks):
    rows, per_tile = _pad_rows(n_nodes + 1)
    nblk = per_tile // CHUNK
    mesh = plsc.VectorSubcoreMesh(core_axis_name="c", subcore_axis_name="s")

    @functools.partial(
        pl.kernel, mesh=mesh,
        out_type=jax.ShapeDtypeStruct((NC, rows, DEG_W), F32),
        scratch_types=[
            pltpu.VMEM((k_chunks, CHUNK), jnp.int32),
            pltpu.VMEM((CHUNK, DEG_W), F32),
            pltpu.VMEM_SHARED((rows, DEG_W), F32),
        ],
    )
    def deg_kernel(dst_hbm, out_hbm, dst_v, ones_v, acc):
        c = lax.axis_index("c")
        s = lax.axis_index("s")

        def zero_row(i, _):
            ones_v[i, :] = jnp.zeros((DEG_W,), F32)
            return 0
        lax.fori_loop(0, CHUNK, zero_row, 0)
        for b in range(nblk):
            pltpu.sync_copy(ones_v, acc.at[pl.ds(s * per_tile + b * CHUNK, CHUNK)])

        def fill(i, _):
            ones_v[i, :] = jnp.ones((DEG_W,), F32)
            return 0
        lax.fori_loop(0, CHUNK, fill, 0)

        pltpu.sync_copy(dst_hbm.at[c, s], dst_v)
        plsc.subcore_barrier()

        def body(j, _):
            pltpu.sync_copy(ones_v, acc.at[dst_v.at[j]], add=True)
            return 0
        lax.fori_loop(0, k_chunks, body, 0)

        plsc.subcore_barrier()
        pltpu.sync_copy(acc.at[pl.ds(s * per_tile, per_tile)],
                        out_hbm.at[c, pl.ds(s * per_tile, per_tile)])

    return deg_kernel


def _sc_agg(n_nodes, d_cols, k_chunks):
    rows, per_tile = _pad_rows(n_nodes + 1)
    nblk = per_tile // CHUNK
    mesh = plsc.VectorSubcoreMesh(core_axis_name="c", subcore_axis_name="s")

    @functools.partial(
        pl.kernel, mesh=mesh,
        out_type=jax.ShapeDtypeStruct((NC, rows, d_cols), F32),
        scratch_types=[
            pltpu.VMEM((k_chunks, CHUNK), jnp.int32),
            pltpu.VMEM((k_chunks, CHUNK), jnp.int32),
            pltpu.VMEM((CHUNK, d_cols), F32),
            pltpu.VMEM_SHARED((rows, d_cols), F32),
            pltpu.SemaphoreType.DMA,
        ],
    )
    def agg_kernel(y_hbm, src_hbm, dst_hbm, out_hbm,
                   src_v, dst_v, rows_v, acc, sem):
        c = lax.axis_index("c")
        s = lax.axis_index("s")

        def zero_row(i, _):
            for cc in range(d_cols // 16):
                rows_v[i, pl.ds(cc * 16, 16)] = jnp.zeros((16,), F32)
            return 0
        lax.fori_loop(0, CHUNK, zero_row, 0)
        for b in range(nblk):
            pltpu.sync_copy(rows_v, acc.at[pl.ds(s * per_tile + b * CHUNK, CHUNK)])

        pltpu.sync_copy(src_hbm.at[c, s], src_v)
        pltpu.sync_copy(dst_hbm.at[c, s], dst_v)
        plsc.subcore_barrier()

        def body(j, _):
            pltpu.async_copy(y_hbm.at[src_v.at[j]], rows_v, sem).wait()
            pltpu.sync_copy(rows_v, acc.at[dst_v.at[j]], add=True)
            return 0
        lax.fori_loop(0, k_chunks, body, 0)

        plsc.subcore_barrier()
        pltpu.sync_copy(acc.at[pl.ds(s * per_tile, per_tile)],
                        out_hbm.at[c, pl.ds(s * per_tile, per_tile)])

    return agg_kernel


_SELU_ALPHA = 1.6732632423543772
_SELU_SCALE = 1.0507009873554805


def _dinv_from_deg(degp_ref):
    deg = degp_ref[0, :, 0:1] + degp_ref[1, :, 0:1] + 1.0
    return lax.rsqrt(deg)


def _tc_y1_body(degp_ref, x_ref, w1_ref, y1_ref):
    dinv = _dinv_from_deg(degp_ref)
    xw = jnp.dot(x_ref[...], w1_ref[...], preferred_element_type=F32)
    y1_ref[...] = dinv * xw


def _tc_mid_body(degp_ref, aggp_ref, y1_ref, b1_ref, w2_ref, y2_ref):
    dinv = _dinv_from_deg(degp_ref)
    su = aggp_ref[0] + aggp_ref[1] + y1_ref[...]
    pre = dinv * su + b1_ref[...][None, :]
    h = _SELU_SCALE * jnp.where(pre > 0, pre, _SELU_ALPHA * (jnp.exp(pre) - 1.0))
    y2 = dinv * jnp.dot(h, w2_ref[...], preferred_element_type=F32)
    y2_ref[...] = jnp.concatenate([y2, jnp.zeros_like(y2)], axis=1)


def _tc_out_body(degp_ref, aggp_ref, y2_ref, b2_ref, o_ref):
    dinv = _dinv_from_deg(degp_ref)
    o_dim = o_ref.shape[1]
    z = (dinv * (aggp_ref[0, :, :o_dim] + aggp_ref[1, :, :o_dim]
                 + y2_ref[:, :o_dim]) + b2_ref[...][None, :])
    m = jnp.max(z, axis=1, keepdims=True)
    e = jnp.exp(z - m)
    o_ref[...] = e / jnp.sum(e, axis=1, keepdims=True)


def kernel(x, edge_index, W1, b1, W2, b2):
    n, d_in = x.shape
    h_dim = W1.shape[1]
    o_dim = W2.shape[1]
    e = edge_index.shape[1]

    k_chunks = -(-e // (NW * CHUNK))
    e_pad = NW * k_chunks * CHUNK
    src = jnp.concatenate(
        [edge_index[0], jnp.zeros((e_pad - e,), jnp.int32)]).reshape(
            NC, NS, k_chunks, CHUNK)
    dst = jnp.concatenate(
        [edge_index[1], jnp.full((e_pad - e,), n, jnp.int32)]).reshape(
            NC, NS, k_chunks, CHUNK)

    degp = _sc_deg(n, k_chunks)(dst)

    rblk = 2000
    grid = (n // rblk,)
    degp_spec = pl.BlockSpec((NC, rblk, DEG_W), lambda i: (0, i, 0))
    aggp_spec = lambda d: pl.BlockSpec((NC, rblk, d), lambda i: (0, i, 0))
    full = lambda *shape: pl.BlockSpec(shape, lambda i: (0,) * len(shape))

    y1 = pl.pallas_call(
        _tc_y1_body,
        grid=grid,
        in_specs=[degp_spec,
                  pl.BlockSpec((rblk, d_in), lambda i: (i, 0)),
                  full(d_in, h_dim)],
        out_specs=pl.BlockSpec((rblk, h_dim), lambda i: (i, 0)),
        out_shape=jax.ShapeDtypeStruct((n, h_dim), F32),
    )(degp, x, W1)

    agg1p = _sc_agg(n, h_dim, k_chunks)(y1, src, dst)

    p2 = 2 * o_dim
    y2 = pl.pallas_call(
        _tc_mid_body,
        grid=grid,
        in_specs=[degp_spec, aggp_spec(h_dim),
                  pl.BlockSpec((rblk, h_dim), lambda i: (i, 0)),
                  full(h_dim), full(h_dim, o_dim)],
        out_specs=pl.BlockSpec((rblk, p2), lambda i: (i, 0)),
        out_shape=jax.ShapeDtypeStruct((n, p2), F32),
    )(degp, agg1p, y1, b1, W2)

    agg2p = _sc_agg(n, p2, k_chunks)(y2, src, dst)

    out = pl.pallas_call(
        _tc_out_body,
        grid=grid,
        in_specs=[degp_spec, aggp_spec(p2),
                  pl.BlockSpec((rblk, p2), lambda i: (i, 0)),
                  full(o_dim)],
        out_specs=pl.BlockSpec((rblk, o_dim), lambda i: (i, 0)),
        out_shape=jax.ShapeDtypeStruct((n, o_dim), F32),
    )(degp, agg2p, y2, b2)

    return out
